# baseline (speedup 1.0000x reference)
import functools

import jax
import jax.numpy as jnp
from jax.experimental import pallas as pl
from jax.experimental.pallas import tpu as pltpu
from jax.experimental.pallas import tpu_sc as plsc

N_TASKS = 1024
N_SPLITS = 8
N_SKILLS = 64
D = N_SPLITS * N_SKILLS
B = 16384
EPS = 1e-12

GATHER_WINDOW = 64
NBUF = 2

NC = 2
NS = 16
NW = NC * NS


def _normalize_body(x_ref, o_ref):
    s = jax.nn.sigmoid(x_ref[...])
    for g in range(N_SPLITS):
        blk = s[:, g * N_SKILLS:(g + 1) * N_SKILLS]
        denom = jnp.sum(blk, axis=-1, keepdims=True) + EPS
        o_ref[:, g * N_SKILLS:(g + 1) * N_SKILLS] = blk / denom


def _normalize_table(module_logits):
    return pl.pallas_call(
        _normalize_body,
        out_shape=jax.ShapeDtypeStruct((N_TASKS, D), jnp.float32),
    )(module_logits)


def _sc_gather(table, idx, batch):
    mesh = plsc.VectorSubcoreMesh(core_axis_name="c", subcore_axis_name="s")
    b_per_w = batch // NW

    @functools.partial(
        pl.kernel,
        out_type=jax.ShapeDtypeStruct((batch * D,), jnp.float32),
        mesh=mesh,
        scratch_types=[
            pltpu.VMEM((b_per_w,), jnp.int32),
            pltpu.VMEM((NBUF, GATHER_WINDOW, D), jnp.float32),
            pltpu.SemaphoreType.DMA((NBUF,)),
            pltpu.SemaphoreType.DMA((NBUF,)),
        ],
    )
    def k(table_hbm, idx_hbm, out_hbm, idx_v, rows_v, gsem, osem):
        wid = jax.lax.axis_index("s") * NC + jax.lax.axis_index("c")
        base = wid * b_per_w
        pltpu.sync_copy(idx_hbm.at[pl.ds(base, b_per_w)], idx_v)

        n = b_per_w // GATHER_WINDOW
        W = GATHER_WINDOW

        def put(c):
            b = c % NBUF
            return [
                pltpu.async_copy(
                    rows_v.at[b, r],
                    out_hbm.at[pl.ds((base + c * W + r) * D, D)],
                    osem.at[b],
                )
                for r in range(W)
            ]

        g = [None] * n
        o = [None] * n
        for c in range(n):
            b = c % NBUF
            if c >= NBUF:
                for h in o[c - NBUF]:
                    h.wait()
            g[c] = pltpu.async_copy(
                table_hbm.at[idx_v.at[pl.ds(c * W, W)]], rows_v.at[b], gsem.at[b]
            )
            if c >= 1:
                g[c - 1].wait()
                o[c - 1] = put(c - 1)
        g[n - 1].wait()
        o[n - 1] = put(n - 1)
        for c in range(max(0, n - NBUF + 1), n):
            for h in o[c]:
                h.wait()

    return k(table, idx)


def kernel(module_logits, task_ids):
    table = _normalize_table(module_logits)
    flat = _sc_gather(table, task_ids.astype(jnp.int32), B)
    return flat.reshape(B, N_SPLITS, N_SKILLS)


# speedup vs baseline: 1.7422x; 1.7422x over previous
import functools

import jax
import jax.numpy as jnp
from jax.experimental import pallas as pl
from jax.experimental.pallas import tpu as pltpu
from jax.experimental.pallas import tpu_sc as plsc

N_TASKS = 1024
N_SPLITS = 8
N_SKILLS = 64
D = N_SPLITS * N_SKILLS
B = 16384
EPS = 1e-12

GATHER_WINDOW = 64
NBUF = 3

NC = 2
NS = 16
NW = NC * NS


def _normalize_body(x_ref, o_ref):
    s = jax.nn.sigmoid(x_ref[...])
    for g in range(N_SPLITS):
        blk = s[:, g * N_SKILLS:(g + 1) * N_SKILLS]
        denom = jnp.sum(blk, axis=-1, keepdims=True) + EPS
        o_ref[:, g * N_SKILLS:(g + 1) * N_SKILLS] = blk / denom


def _normalize_table(module_logits):
    return pl.pallas_call(
        _normalize_body,
        out_shape=jax.ShapeDtypeStruct((N_TASKS, D), jnp.float32),
    )(module_logits)


def _sc_gather(table, idx, batch):
    mesh = plsc.VectorSubcoreMesh(core_axis_name="c", subcore_axis_name="s")
    b_per_w = batch // NW

    @functools.partial(
        pl.kernel,
        out_type=jax.ShapeDtypeStruct((batch, D), jnp.float32),
        mesh=mesh,
        scratch_types=[
            pltpu.VMEM((b_per_w,), jnp.int32),
            pltpu.VMEM((NBUF, GATHER_WINDOW, D), jnp.float32),
            pltpu.SemaphoreType.DMA((NBUF,)),
            pltpu.SemaphoreType.DMA((NBUF,)),
        ],
    )
    def k(table_hbm, idx_hbm, out_hbm, idx_v, rows_v, gsem, osem):
        wid = jax.lax.axis_index("s") * NC + jax.lax.axis_index("c")
        base = wid * b_per_w
        pltpu.sync_copy(idx_hbm.at[pl.ds(base, b_per_w)], idx_v)

        n = b_per_w // GATHER_WINDOW
        W = GATHER_WINDOW
        g = [None] * n
        o = [None] * n
        for c in range(n):
            b = c % NBUF
            if c >= NBUF:
                o[c - NBUF].wait()
            g[c] = pltpu.async_copy(
                table_hbm.at[idx_v.at[pl.ds(c * W, W)]], rows_v.at[b], gsem.at[b]
            )
            if c >= 1:
                g[c - 1].wait()
                o[c - 1] = pltpu.async_copy(
                    rows_v.at[(c - 1) % NBUF],
                    out_hbm.at[pl.ds(base + (c - 1) * W, W)],
                    osem.at[(c - 1) % NBUF],
                )
        g[n - 1].wait()
        o[n - 1] = pltpu.async_copy(
            rows_v.at[(n - 1) % NBUF], out_hbm.at[pl.ds(base + (n - 1) * W, W)],
            osem.at[(n - 1) % NBUF],
        )
        for c in range(max(0, n - NBUF + 1), n):
            o[c].wait()

    return k(table, idx)


def kernel(module_logits, task_ids):
    table = _normalize_table(module_logits)
    flat = _sc_gather(table, task_ids.astype(jnp.int32), B)
    return flat.reshape(B, N_SPLITS, N_SKILLS)
